# baseline (device time: 9369 ns/iter reference)
import jax
import jax.numpy as jnp
from jax import lax
from jax.experimental import pallas as pl
from jax.experimental.pallas import tpu as pltpu

N_GLOBAL = 2048
NB = 8
NBUF = 4


def kernel(x):
    m_per, n_per = x.shape
    bm = m_per // NB
    rows = bm // 128
    inv = 1.0 / N_GLOBAL

    def body(x_hbm, out_ref, bufs, recv_buf, cp_sems, send_sem, recv_sem):
        my_x = lax.axis_index("x")
        my_y = lax.axis_index("y")
        nbr = (my_x, 1 - my_y)

        barrier_sem = pltpu.get_barrier_semaphore()
        pl.semaphore_signal(
            barrier_sem, inc=1, device_id=nbr,
            device_id_type=pl.DeviceIdType.MESH,
        )

        def cp(k):
            return pltpu.make_async_copy(
                x_hbm.at[pl.ds(k * bm, bm), :],
                bufs.at[k % NBUF],
                cp_sems.at[k],
            )

        for k in range(NBUF):
            cp(k).start()
        for k in range(NB):
            cp(k).wait()
            p = jnp.sum(bufs[k % NBUF], axis=1)
            if k + NBUF < NB:
                cp(k + NBUF).start()
            out_ref[pl.ds(k * rows, rows), :] = p.reshape(rows, 128)

        pl.semaphore_wait(barrier_sem, 1)
        rdma = pltpu.make_async_remote_copy(
            src_ref=out_ref,
            dst_ref=recv_buf,
            send_sem=send_sem,
            recv_sem=recv_sem,
            device_id=nbr,
            device_id_type=pl.DeviceIdType.MESH,
        )
        rdma.start()
        rdma.wait()
        out_ref[:, :] = (out_ref[:, :] + recv_buf[:, :]) * inv

    res = pl.pallas_call(
        body,
        out_shape=jax.ShapeDtypeStruct((NB * rows, 128), jnp.float32),
        in_specs=[pl.BlockSpec(memory_space=pltpu.MemorySpace.HBM)],
        out_specs=pl.BlockSpec(memory_space=pltpu.VMEM),
        scratch_shapes=[
            pltpu.VMEM((NBUF, bm, n_per), jnp.float32),
            pltpu.VMEM((NB * rows, 128), jnp.float32),
            pltpu.SemaphoreType.DMA((NB,)),
            pltpu.SemaphoreType.DMA,
            pltpu.SemaphoreType.DMA,
        ],
        compiler_params=pltpu.CompilerParams(collective_id=0),
    )(x)
    return jnp.reshape(res, (m_per, 1))


# device time: 8440 ns/iter; 1.1101x vs baseline; 1.1101x over previous
import jax
import jax.numpy as jnp
from jax import lax
from jax.experimental import pallas as pl
from jax.experimental.pallas import tpu as pltpu

N_GLOBAL = 2048
G = 4


def kernel(x):
    m_per, n_per = x.shape
    bm = m_per // (2 * G)
    rows = bm // 128
    inv = 1.0 / N_GLOBAL

    def body(xa_ref, xb_ref, out_ref, send_buf, recv_buf, send_sem, recv_sem):
        i = pl.program_id(0)
        my_x = lax.axis_index("x")
        my_y = lax.axis_index("y")
        nbr = (my_x, 1 - my_y)

        barrier_sem = pltpu.get_barrier_semaphore()

        pa = jnp.sum(xa_ref[:, :], axis=1)
        pb = jnp.sum(xb_ref[:, :], axis=1)
        send_buf[pl.ds((2 * i) * rows, rows), :] = pa.reshape(rows, 128)
        send_buf[pl.ds((2 * i + 1) * rows, rows), :] = pb.reshape(rows, 128)

        @pl.when(i == 0)
        def _():
            pl.semaphore_signal(
                barrier_sem, inc=1, device_id=nbr,
                device_id_type=pl.DeviceIdType.MESH,
            )

        @pl.when(i == G - 1)
        def _():
            pl.semaphore_wait(barrier_sem, 1)
            rdma = pltpu.make_async_remote_copy(
                src_ref=send_buf,
                dst_ref=recv_buf,
                send_sem=send_sem,
                recv_sem=recv_sem,
                device_id=nbr,
                device_id_type=pl.DeviceIdType.MESH,
            )
            rdma.start()
            rdma.wait()
            out_ref[:, :] = (send_buf[:, :] + recv_buf[:, :]) * inv

    res = pl.pallas_call(
        body,
        grid=(G,),
        out_shape=jax.ShapeDtypeStruct((2 * G * rows, 128), jnp.float32),
        in_specs=[
            pl.BlockSpec((bm, n_per), lambda i: (2 * i, 0),
                         memory_space=pltpu.VMEM),
            pl.BlockSpec((bm, n_per), lambda i: (2 * i + 1, 0),
                         memory_space=pltpu.VMEM),
        ],
        out_specs=pl.BlockSpec((2 * G * rows, 128), lambda i: (0, 0),
                               memory_space=pltpu.VMEM),
        scratch_shapes=[
            pltpu.VMEM((2 * G * rows, 128), jnp.float32),
            pltpu.VMEM((2 * G * rows, 128), jnp.float32),
            pltpu.SemaphoreType.DMA,
            pltpu.SemaphoreType.DMA,
        ],
        compiler_params=pltpu.CompilerParams(collective_id=0),
    )(x, x)
    return jnp.reshape(res, (m_per, 1))


# device time: 8283 ns/iter; 1.1311x vs baseline; 1.0190x over previous
import jax
import jax.numpy as jnp
from jax import lax
from jax.experimental import pallas as pl
from jax.experimental.pallas import tpu as pltpu

N_GLOBAL = 2048
G = 2


def kernel(x):
    m_per, n_per = x.shape
    bm = m_per // G
    rows = bm // 128
    inv = 1.0 / N_GLOBAL

    def body(x_ref, out_ref, send_buf, recv_buf, send_sem, recv_sem):
        i = pl.program_id(0)
        my_x = lax.axis_index("x")
        my_y = lax.axis_index("y")
        nbr = (my_x, 1 - my_y)

        barrier_sem = pltpu.get_barrier_semaphore()

        p = jnp.sum(x_ref[:, :], axis=1)
        send_buf[pl.ds(i * rows, rows), :] = p.reshape(rows, 128)

        @pl.when(i == 0)
        def _():
            pl.semaphore_signal(
                barrier_sem, inc=1, device_id=nbr,
                device_id_type=pl.DeviceIdType.MESH,
            )

        @pl.when(i == G - 1)
        def _():
            pl.semaphore_wait(barrier_sem, 1)
            rdma = pltpu.make_async_remote_copy(
                src_ref=send_buf,
                dst_ref=recv_buf,
                send_sem=send_sem,
                recv_sem=recv_sem,
                device_id=nbr,
                device_id_type=pl.DeviceIdType.MESH,
            )
            rdma.start()
            rdma.wait()
            out_ref[:, :] = (send_buf[:, :] + recv_buf[:, :]) * inv

    res = pl.pallas_call(
        body,
        grid=(G,),
        out_shape=jax.ShapeDtypeStruct((G * rows, 128), jnp.float32),
        in_specs=[
            pl.BlockSpec((bm, n_per), lambda i: (i, 0),
                         memory_space=pltpu.VMEM),
        ],
        out_specs=pl.BlockSpec((G * rows, 128), lambda i: (0, 0),
                               memory_space=pltpu.VMEM),
        scratch_shapes=[
            pltpu.VMEM((G * rows, 128), jnp.float32),
            pltpu.VMEM((G * rows, 128), jnp.float32),
            pltpu.SemaphoreType.DMA,
            pltpu.SemaphoreType.DMA,
        ],
        compiler_params=pltpu.CompilerParams(collective_id=0),
    )(x)
    return jnp.reshape(res, (m_per, 1))
